# baseline (speedup 1.0000x reference)
import functools
from itertools import combinations

import jax
import jax.numpy as jnp
from jax import lax
from jax.experimental import pallas as pl
from jax.experimental.pallas import tpu as pltpu
from jax.experimental.pallas import tpu_sc as plsc

B, F, D = 16384, 26, 16
P = F * (F - 1) // 2
BC = 1024

_PAIRS = list(combinations(range(F), 2))

NW = 32
BW = B // NW
CB = 256


def _stats_body(xt_ref, prod_ref, s_ref):
    i = pl.program_id(0)
    s1_parts, s2_parts = [], []
    hs, row0 = [], 0
    for fi, gj in _PAIRS:
        q = xt_ref[fi * D:(fi + 1) * D, :] * xt_ref[gj * D:(gj + 1) * D, :]
        hs.append(q[0:8] + q[8:16])
        if len(hs) == 8 or row0 + len(hs) == P:
            n = len(hs)
            H = jnp.stack(hs, axis=0)
            S = jnp.sum(jnp.swapaxes(H, 0, 1), axis=0)
            prod_ref[row0:row0 + n, :] = S
            s1_parts.append(jnp.sum(S, axis=1, keepdims=True))
            s2_parts.append(jnp.sum(S * S, axis=1, keepdims=True))
            hs = []
            row0 += n
    s1 = jnp.concatenate(s1_parts, axis=0)
    s2 = jnp.concatenate(s2_parts, axis=0)
    s = jnp.concatenate([s1, s2], axis=1)

    @pl.when(i == 0)
    def _():
        s_ref[...] = jnp.zeros_like(s_ref)

    s_ref[...] += s


def _finalize_body(s_ref, alpha_ref, w_ref):
    s = s_ref[...]
    m = s[:, 0:1] * (1.0 / B)
    var = s[:, 1:2] * (1.0 / B) - m * m
    w = jnp.tanh(alpha_ref[...]) * jax.lax.rsqrt(var + 1e-3)
    c = jnp.sum(w * m)
    ones16 = jnp.ones((1, 16), jnp.float32)
    w_ref[...] = jnp.concatenate(
        [w * ones16, jnp.full((1, 16), c, jnp.float32),
         jnp.zeros((336 - P - 1, 16), jnp.float32)], axis=0)


def _sc_out_kernel(prod_hbm, w_hbm, out_hbm, pbuf, wbuf, obuf, sem):
    wid = lax.axis_index("s") * 2 + lax.axis_index("c")
    base = wid * BW
    pltpu.sync_copy(w_hbm, wbuf)
    crow = wbuf[P, :]
    for cb in range(BW // CB):
        col0 = base + cb * CB
        pltpu.async_copy(
            prod_hbm.at[:, pl.ds(col0, CB)], pbuf, sem).wait()

        def body(p, accs):
            wrow = wbuf[p, :]
            return tuple(
                accs[v] + wrow * pbuf[p, pl.ds(v * 16, 16)]
                for v in range(CB // 16))

        accs = tuple(jnp.zeros((16,), jnp.float32) for _ in range(CB // 16))
        accs = lax.fori_loop(0, P, body, accs)
        for v in range(CB // 16):
            obuf[pl.ds(v * 16, 16)] = accs[v] - crow
        pltpu.sync_copy(obuf, out_hbm.at[pl.ds(col0, CB)])


def kernel(embed_matrix, alpha, feat_i, feat_j):
    del feat_i, feat_j
    xt = embed_matrix.reshape(B, F * D).T
    nb = B // BC
    prod, s = pl.pallas_call(
        _stats_body,
        grid=(nb,),
        in_specs=[pl.BlockSpec((F * D, BC), lambda i: (0, i))],
        out_specs=[
            pl.BlockSpec((P, BC), lambda i: (0, i)),
            pl.BlockSpec((P, 2), lambda i: (0, 0)),
        ],
        out_shape=[
            jax.ShapeDtypeStruct((P, B), jnp.float32),
            jax.ShapeDtypeStruct((P, 2), jnp.float32),
        ],
    )(xt)
    wmat = pl.pallas_call(
        _finalize_body,
        in_specs=[
            pl.BlockSpec((P, 2), lambda: (0, 0)),
            pl.BlockSpec((P, 1), lambda: (0, 0)),
        ],
        out_specs=pl.BlockSpec((336, 16), lambda: (0, 0)),
        out_shape=jax.ShapeDtypeStruct((336, 16), jnp.float32),
    )(s, alpha.reshape(P, 1))

    sc_out = functools.partial(
        pl.kernel,
        out_type=jax.ShapeDtypeStruct((B,), jnp.float32),
        mesh=plsc.VectorSubcoreMesh(core_axis_name="c", subcore_axis_name="s"),
        scratch_types=[
            pltpu.VMEM((P, CB), jnp.float32),
            pltpu.VMEM((336, 16), jnp.float32),
            pltpu.VMEM((CB,), jnp.float32),
            pltpu.SemaphoreType.DMA,
        ],
    )(_sc_out_kernel)
    out = sc_out(prod, wmat)
    return out.reshape(B, 1)


# speedup vs baseline: 1.3534x; 1.3534x over previous
from itertools import combinations

import jax
import jax.numpy as jnp
from jax.experimental import pallas as pl

B, F, D = 16384, 26, 16
P = F * (F - 1) // 2
BC = 1024

_PAIRS = list(combinations(range(F), 2))


def _stats_body(xt_ref, prod_ref, s_ref):
    i = pl.program_id(0)
    s1_parts, s2_parts = [], []
    hs, row0 = [], 0
    for fi, gj in _PAIRS:
        q = xt_ref[fi * D:(fi + 1) * D, :] * xt_ref[gj * D:(gj + 1) * D, :]
        hs.append(q[0:8] + q[8:16])
        if len(hs) == 8 or row0 + len(hs) == P:
            n = len(hs)
            H = jnp.stack(hs, axis=0)
            S = jnp.sum(jnp.swapaxes(H, 0, 1), axis=0)
            prod_ref[row0:row0 + n, :] = S
            s1_parts.append(jnp.sum(S, axis=1, keepdims=True))
            s2_parts.append(jnp.sum(S * S, axis=1, keepdims=True))
            hs = []
            row0 += n
    s1 = jnp.concatenate(s1_parts, axis=0)
    s2 = jnp.concatenate(s2_parts, axis=0)
    s = jnp.concatenate([s1, s2], axis=1)

    @pl.when(i == 0)
    def _():
        s_ref[...] = jnp.zeros_like(s_ref)

    s_ref[...] += s


def _out_body(s_ref, alpha_ref, prod_ref, out_ref):
    s = s_ref[...]
    m = s[:, 0:1] * (1.0 / B)
    var = s[:, 1:2] * (1.0 / B) - m * m
    w = jnp.tanh(alpha_ref[...]) * jax.lax.rsqrt(var + 1e-3)
    c = jnp.sum(w * m)
    out_ref[...] = jnp.sum(prod_ref[...] * w, axis=0, keepdims=True) - c


def kernel(embed_matrix, alpha, feat_i, feat_j):
    del feat_i, feat_j
    xt = embed_matrix.reshape(B, F * D).T
    nb = B // BC
    prod, s = pl.pallas_call(
        _stats_body,
        grid=(nb,),
        in_specs=[pl.BlockSpec((F * D, BC), lambda i: (0, i))],
        out_specs=[
            pl.BlockSpec((P, BC), lambda i: (0, i)),
            pl.BlockSpec((P, 2), lambda i: (0, 0)),
        ],
        out_shape=[
            jax.ShapeDtypeStruct((P, B), jnp.float32),
            jax.ShapeDtypeStruct((P, 2), jnp.float32),
        ],
    )(xt)
    out = pl.pallas_call(
        _out_body,
        grid=(nb,),
        in_specs=[
            pl.BlockSpec((P, 2), lambda i: (0, 0)),
            pl.BlockSpec((P, 1), lambda i: (0, 0)),
            pl.BlockSpec((P, BC), lambda i: (0, i)),
        ],
        out_specs=pl.BlockSpec((1, BC), lambda i: (0, i)),
        out_shape=jax.ShapeDtypeStruct((1, B), jnp.float32),
    )(s, alpha.reshape(P, 1), prod)
    return out.reshape(B, 1)
